# baseline (speedup 1.0000x reference)
import functools

import jax
import jax.numpy as jnp
from jax import lax
from jax.experimental import pallas as pl
from jax.experimental.pallas import tpu as pltpu, tpu_sc as plsc

CHUNK = 128
NSLOT = 6
LOOK = 3
PUT_GROUPS = ((0, 1), (2, 3), (4, 5))


@jax.jit
def _lookup(idx, table):
    info = plsc.get_sparse_core_info()
    nc, ns = info.num_cores, info.num_subcores
    nw = nc * ns
    n = idx.shape[0]
    d = table.shape[1]
    per_w = n // nw
    n_chunks = per_w // CHUNK
    idx3 = idx.reshape(nw, n_chunks, CHUNK)

    mesh = plsc.VectorSubcoreMesh(core_axis_name="c", subcore_axis_name="s")

    @functools.partial(
        pl.kernel,
        mesh=mesh,
        out_type=jax.ShapeDtypeStruct((n // CHUNK, CHUNK, d), jnp.float32),
        scratch_types=[
            pltpu.VMEM((n_chunks, CHUNK), jnp.int32),
            pltpu.VMEM((NSLOT, CHUNK, d), jnp.float32),
            [pltpu.SemaphoreType.DMA for _ in range(NSLOT)],
            [pltpu.SemaphoreType.DMA for _ in range(len(PUT_GROUPS))],
        ],
    )
    def gather_kernel(idx_hbm, table_hbm, out_hbm, idx_v, buf, gsems, wsems):
        wid = lax.axis_index("s") * nc + lax.axis_index("c")
        pltpu.sync_copy(idx_hbm.at[wid], idx_v)
        base = wid * n_chunks

        def gather_start(j):
            pltpu.async_copy(table_hbm.at[idx_v.at[j]],
                             buf.at[j % NSLOT], gsems[j % NSLOT])

        def gather_wait(j):
            pltpu.make_async_copy(table_hbm.at[idx_v.at[j]],
                                  buf.at[j % NSLOT], gsems[j % NSLOT]).wait()

        def put_pair(g, j_first, k):
            slots = PUT_GROUPS[k]
            src = buf.at[pl.ds(slots[0], len(slots))]
            dst = out_hbm.at[pl.ds(base + j_first, len(slots))]
            return src, dst, wsems[k]

        pending = {}
        to_drain = []

        def put_start(g, j_first, k):
            src, dst, sem = put_pair(g, j_first, k)
            pltpu.async_copy(src, dst, sem)
            rec = [(src, dst, sem), False]
            for s in PUT_GROUPS[k]:
                pending[s] = rec
            to_drain.append(rec)

        def reclaim(slot):
            rec = pending.get(slot)
            if rec is not None and not rec[1]:
                (src, dst, sem), _ = rec
                pltpu.make_async_copy(src, dst, sem).wait()
                rec[1] = True

        for j in range(LOOK):
            gather_start(j)

        for j in range(n_chunks):
            if j + LOOK < n_chunks:
                reclaim((j + LOOK) % NSLOT)
                gather_start(j + LOOK)
            gather_wait(j)
            b = j % NSLOT
            for k, slots in enumerate(PUT_GROUPS):
                if b == slots[-1]:
                    put_start(j // NSLOT, j - len(slots) + 1, k)

        for rec in to_drain:
            if not rec[1]:
                (src, dst, sem), _ = rec
                pltpu.make_async_copy(src, dst, sem).wait()
                rec[1] = True

    out = gather_kernel(idx3, table)
    return out.reshape(n, d)


def kernel(batch, table):
    b, h = batch.shape
    d = table.shape[1]
    idx = batch.reshape(-1).astype(jnp.int32)
    out = _lookup(idx, table)
    return out.reshape(b, h, d)


# speedup vs baseline: 1.0115x; 1.0115x over previous
import functools

import jax
import jax.numpy as jnp
from jax import lax
from jax.experimental import pallas as pl
from jax.experimental.pallas import tpu as pltpu, tpu_sc as plsc

CHUNK = 128
NBUF = 5


@jax.jit
def _lookup(idx, table):
    info = plsc.get_sparse_core_info()
    nc, ns = info.num_cores, info.num_subcores
    nw = nc * ns
    n = idx.shape[0]
    d = table.shape[1]
    per_w = n // nw
    n_chunks = per_w // CHUNK
    n_groups = n_chunks // NBUF
    idx3 = idx.reshape(nw, n_chunks, CHUNK)

    mesh = plsc.VectorSubcoreMesh(core_axis_name="c", subcore_axis_name="s")

    @functools.partial(
        pl.kernel,
        mesh=mesh,
        out_type=jax.ShapeDtypeStruct((n, d), jnp.float32),
        scratch_types=[
            pltpu.VMEM((n_chunks, CHUNK), jnp.int32),
            [pltpu.VMEM((CHUNK, d), jnp.float32) for _ in range(NBUF)],
            [pltpu.SemaphoreType.DMA for _ in range(NBUF)],
            [pltpu.SemaphoreType.DMA for _ in range(NBUF)],
        ],
    )
    def gather_kernel(idx_hbm, table_hbm, out_hbm, idx_v, bufs, gsems, wsems):
        wid = lax.axis_index("s") * nc + lax.axis_index("c")
        base = wid * per_w

        def gather_start(j, b):
            pltpu.async_copy(table_hbm.at[idx_v.at[j]], bufs[b], gsems[b])

        def gather_wait(j, b):
            pltpu.make_async_copy(
                table_hbm.at[idx_v.at[j]], bufs[b], gsems[b]).wait()

        def out_slab(j):
            return out_hbm.at[pl.ds(base + j * CHUNK, CHUNK)]

        def put_start(j, b):
            pltpu.async_copy(bufs[b], out_slab(j), wsems[b])

        def put_wait(j, b):
            pltpu.make_async_copy(bufs[b], out_slab(j), wsems[b]).wait()

        LOOK = NBUF - 2

        pre = 8
        pltpu.sync_copy(idx_hbm.at[wid, pl.ds(0, pre)],
                        idx_v.at[pl.ds(0, pre)])
        for b in range(LOOK):
            gather_start(b, b)
        pltpu.sync_copy(idx_hbm.at[wid, pl.ds(pre, n_chunks - pre)],
                        idx_v.at[pl.ds(pre, n_chunks - pre)])

        def body(i, carry):
            j0 = i * NBUF
            for b in range(NBUF):
                j = j0 + b
                s = (b + LOOK) % NBUF

                @pl.when(jnp.logical_and(j >= NBUF - LOOK,
                                         j + LOOK < n_chunks))
                def _():
                    put_wait(j - (NBUF - LOOK), s)

                @pl.when(j + LOOK < n_chunks)
                def _():
                    gather_start(j + LOOK, s)

                gather_wait(j, b)
                put_start(j, b)
            return carry

        lax.fori_loop(0, n_groups, body, 0)

        for b in range(NBUF):
            j = n_chunks - NBUF + b
            put_wait(j, j % NBUF)

    return gather_kernel(idx3, table)


def kernel(batch, table):
    b, h = batch.shape
    d = table.shape[1]
    idx = batch.reshape(-1).astype(jnp.int32)
    out = _lookup(idx, table)
    return out.reshape(b, h, d)
